# baseline (speedup 1.0000x reference)
import functools

import jax
import jax.numpy as jnp
from jax import lax
from jax.experimental import pallas as pl
from jax.experimental.pallas import tpu as pltpu
from jax.experimental.pallas import tpu_sc as plsc

H = 128
NTOK = 32768
CHUNK = 256
NCHUNK = 4
SEQ = 8192
EPS = 1e-12


_GDN = lax.GatherDimensionNumbers(
    offset_dims=(), collapsed_slice_dims=(0,), start_index_map=(0,))


def _vgather(v, idx):
    return lax.gather(v, idx[:, None], _GDN, slice_sizes=(1,),
                      mode=lax.GatherScatterMode.PROMISE_IN_BOUNDS)


def _allsum(v, iot):
    for sh in (1, 2, 4, 8):
        v = v + _vgather(v, iot ^ sh)
    return v


def _rsqrt_nr(x):
    i = lax.bitcast_convert_type(x, jnp.int32)
    i = jnp.int32(0x5F3759DF) - lax.shift_right_logical(i, 1)
    y = lax.bitcast_convert_type(i, jnp.float32)
    for _ in range(3):
        y = y * (1.5 - 0.5 * x * y * y)
    return y


def _sc_embed(ids2d, tt2d, word_emb, pos_emb, type_emb, gamma, beta):
    info = plsc.get_sparse_core_info()
    nc, ns = info.num_cores, info.num_subcores
    nw = nc * ns
    tok_per_w = NTOK // nw
    rows_per_chunk = CHUNK // H

    mesh = plsc.VectorSubcoreMesh(core_axis_name="c", subcore_axis_name="s")

    @functools.partial(
        pl.kernel,
        out_type=jax.ShapeDtypeStruct((NTOK, H), jnp.float32),
        mesh=mesh,
        scratch_types=[
            pltpu.VMEM((tok_per_w // H, H), jnp.int32),
            pltpu.VMEM((tok_per_w // H, H), jnp.int32),
            pltpu.VMEM((CHUNK, H), jnp.float32),
            pltpu.VMEM((CHUNK, H), jnp.float32),
            pltpu.VMEM((CHUNK, H), jnp.float32),
            pltpu.VMEM((H,), jnp.float32),
            pltpu.VMEM((H,), jnp.float32),
            pltpu.SemaphoreType.DMA,
        ],
    )
    def k(ids_hbm, tt_hbm, word_hbm, pos_hbm, type_hbm, g_hbm, b_hbm,
          out_hbm, idx_v, ttx_v, rows_v, pos_v, te_v, g_v, b_v, sem):
        wid = lax.axis_index("s") * nc + lax.axis_index("c")

        pltpu.sync_copy(g_hbm, g_v)
        pltpu.sync_copy(b_hbm, b_v)

        iot = lax.iota(jnp.int32, 16)
        gs = [g_v[pl.ds(16 * j, 16)] for j in range(8)]
        bs = [b_v[pl.ds(16 * j, 16)] for j in range(8)]

        wrow0 = wid * (tok_per_w // H)
        pltpu.sync_copy(ids_hbm.at[pl.ds(wrow0, tok_per_w // H)], idx_v)
        pltpu.sync_copy(tt_hbm.at[pl.ds(wrow0, tok_per_w // H)], ttx_v)

        for chunk in range(NCHUNK):
            toff = wid * tok_per_w + chunk * CHUNK
            soff = lax.rem(toff, SEQ)

            pltpu.sync_copy(pos_hbm.at[pl.ds(soff, CHUNK)], pos_v)

            cps = []
            for r in range(rows_per_chunk):
                cps.append(pltpu.async_copy(
                    word_hbm.at[idx_v.at[chunk * rows_per_chunk + r]],
                    rows_v.at[pl.ds(r * H, H)], sem))
                cps.append(pltpu.async_copy(
                    type_hbm.at[ttx_v.at[chunk * rows_per_chunk + r]],
                    te_v.at[pl.ds(r * H, H)], sem))
            for cp in cps:
                cp.wait()

            def tok(t, carry):
                xs = [rows_v[t, pl.ds(16 * j, 16)]
                      + pos_v[t, pl.ds(16 * j, 16)]
                      + te_v[t, pl.ds(16 * j, 16)] for j in range(8)]
                s1 = ((xs[0] + xs[1]) + (xs[2] + xs[3])) \
                    + ((xs[4] + xs[5]) + (xs[6] + xs[7]))
                sq = [x * x for x in xs]
                s2 = ((sq[0] + sq[1]) + (sq[2] + sq[3])) \
                    + ((sq[4] + sq[5]) + (sq[6] + sq[7]))
                tot1 = _allsum(s1, iot)
                tot2 = _allsum(s2, iot)
                mean = tot1 * (1.0 / H)
                var = tot2 * (1.0 / H) - mean * mean
                r = _rsqrt_nr(var + EPS)
                for j in range(8):
                    rows_v[t, pl.ds(16 * j, 16)] = \
                        (xs[j] - mean) * r * gs[j] + bs[j]
                return carry

            lax.fori_loop(0, CHUNK, tok, 0)

            pltpu.sync_copy(rows_v, out_hbm.at[pl.ds(toff, CHUNK)])

    return k(ids2d, tt2d, word_emb, pos_emb, type_emb, gamma, beta)


def kernel(input_ids, token_type_ids, word_emb, pos_emb, type_emb, gamma, beta):
    b, s = input_ids.shape
    ids2d = input_ids.reshape(-1).astype(jnp.int32).reshape(NTOK // H, H)
    tt2d = token_type_ids.reshape(-1).astype(jnp.int32).reshape(NTOK // H, H)
    out = _sc_embed(ids2d, tt2d, word_emb.astype(jnp.float32),
                    pos_emb.astype(jnp.float32), type_emb.astype(jnp.float32),
                    gamma.astype(jnp.float32), beta.astype(jnp.float32))
    return out.reshape(b, s, H)
